# Initial kernel scaffold; baseline (speedup 1.0000x reference)
import jax
import jax.numpy as jnp
from jax.experimental import pallas as pl


def kernel(x, Wg, bg, W1, b1, W2, b2):
    raise NotImplementedError("write your pallas kernel here")


# speedup vs baseline: 1.5016x; 1.5016x over previous
import jax
import jax.numpy as jnp
from jax import lax
from jax.experimental import pallas as pl
from jax.experimental.pallas import tpu as pltpu

B_, S_, D_, H_, E_, K_ = 2, 2048, 1024, 4096, 8, 2
T_ = B_ * S_
TK_ = T_ * K_
BT_ = 512
MAXB_ = TK_ // BT_ + E_
NPAD_ = MAXB_ * BT_
NH_ = 4
HT_ = H_ // NH_
TBG_ = 512
EP_ = 128


def _gate_body(x_ref, wg_ref, bgm_ref, idx_ref, w_ref):
    s = jnp.dot(x_ref[...].astype(jnp.bfloat16),
                wg_ref[...].astype(jnp.bfloat16),
                preferred_element_type=jnp.float32)
    s = s + bgm_ref[...]
    iota = lax.broadcasted_iota(jnp.int32, s.shape, 1)
    big = jnp.int32(1 << 30)
    m1 = jnp.max(s, axis=1, keepdims=True)
    a1 = jnp.min(jnp.where(s >= m1, iota, big), axis=1, keepdims=True)
    s2 = jnp.where(iota == a1, -1e30, s)
    m2 = jnp.max(s2, axis=1, keepdims=True)
    a2 = jnp.min(jnp.where(s2 >= m2, iota, big), axis=1, keepdims=True)
    e2 = jnp.exp(m2 - m1)
    w1 = 1.0 / (1.0 + e2)
    w2 = e2 / (1.0 + e2)
    idx_ref[...] = jnp.where(iota == 0, a1,
                             jnp.where(iota == 1, a2, 0)).astype(jnp.int32)
    w_ref[...] = jnp.where(iota == 0, w1, jnp.where(iota == 1, w2, 0.0))


def _gate(x2d, wg_pad, bg_pad):
    return pl.pallas_call(
        _gate_body,
        grid=(T_ // TBG_,),
        in_specs=[
            pl.BlockSpec((TBG_, D_), lambda i: (i, 0)),
            pl.BlockSpec((D_, EP_), lambda i: (0, 0)),
            pl.BlockSpec((1, EP_), lambda i: (0, 0)),
        ],
        out_specs=[
            pl.BlockSpec((TBG_, EP_), lambda i: (i, 0)),
            pl.BlockSpec((TBG_, EP_), lambda i: (i, 0)),
        ],
        out_shape=[
            jax.ShapeDtypeStruct((T_, EP_), jnp.int32),
            jax.ShapeDtypeStruct((T_, EP_), jnp.float32),
        ],
    )(x2d, wg_pad, bg_pad)


def _ffn_body(be_ref, bv_ref, xs_ref, w1_ref, b1_ref, w2_ref, b2_ref,
              ws_ref, out_ref):
    h = pl.program_id(1)
    i = pl.program_id(0)

    @pl.when(bv_ref[i] == 1)
    def _():
        @pl.when(h == 0)
        def _():
            out_ref[...] = jnp.zeros_like(out_ref)

        hh = jnp.dot(xs_ref[...], w1_ref[0],
                     preferred_element_type=jnp.float32) + b1_ref[0, 0]
        hb = jnp.maximum(hh, 0.0)
        out_ref[...] += jnp.dot(hb, w2_ref[0],
                                preferred_element_type=jnp.float32)

        @pl.when(h == NH_ - 1)
        def _():
            out_ref[...] = (out_ref[...] + b2_ref[0]) * ws_ref[...]


def _ffn(blk_e, blk_v, xs, W1, b1, W2, b2, ws_pad):
    grid_spec = pltpu.PrefetchScalarGridSpec(
        num_scalar_prefetch=2,
        grid=(MAXB_, NH_),
        in_specs=[
            pl.BlockSpec((BT_, D_), lambda i, h, be, bv: (i, 0)),
            pl.BlockSpec((1, D_, HT_),
                         lambda i, h, be, bv: (be[i], 0, h * bv[i])),
            pl.BlockSpec((1, 1, 1, HT_),
                         lambda i, h, be, bv: (be[i], h * bv[i], 0, 0)),
            pl.BlockSpec((1, HT_, D_),
                         lambda i, h, be, bv: (be[i], h * bv[i], 0)),
            pl.BlockSpec((1, 1, D_), lambda i, h, be, bv: (be[i], 0, 0)),
            pl.BlockSpec((BT_, 1), lambda i, h, be, bv: (i, 0)),
        ],
        out_specs=pl.BlockSpec((BT_, D_), lambda i, h, be, bv: (i, 0)),
    )
    return pl.pallas_call(
        _ffn_body,
        grid_spec=grid_spec,
        out_shape=jax.ShapeDtypeStruct((NPAD_, D_), jnp.float32),
    )(blk_e, blk_v, xs, W1, b1.reshape(E_, NH_, 1, HT_), W2,
      b2.reshape(E_, 1, D_), ws_pad)


def kernel(x, Wg, bg, W1, b1, W2, b2):
    x2d = x.reshape(T_, D_)
    wg_pad = jnp.zeros((D_, EP_), jnp.float32).at[:, :E_].set(Wg)
    bg_pad = jnp.full((1, EP_), -1e30, jnp.float32).at[0, :E_].set(bg)
    idx128, w128 = _gate(x2d, wg_pad, bg_pad)
    e_flat = idx128[:, :K_].reshape(TK_)
    w_flat = w128[:, :K_].reshape(TK_)

    a_iota = jnp.arange(TK_, dtype=jnp.int32)
    skeys = jnp.sort(e_flat * TK_ + a_iota)
    order = skeys % TK_
    e_sorted = skeys // TK_
    tok_sorted = order // K_
    w_sorted = w_flat[order]

    counts = jnp.bincount(e_flat, length=E_).astype(jnp.int32)
    starts = jnp.concatenate(
        [jnp.zeros((1,), jnp.int32), jnp.cumsum(counts)[:-1]])
    nbk = (counts + BT_ - 1) // BT_
    prow = jnp.concatenate(
        [jnp.zeros((1,), jnp.int32), jnp.cumsum(nbk * BT_)[:-1]])
    ppos = prow[e_sorted] + (a_iota - starts[e_sorted])

    tok_padded = jnp.zeros((NPAD_,), jnp.int32).at[ppos].set(tok_sorted)
    ws_padded = jnp.zeros((NPAD_,), jnp.float32).at[ppos].set(w_sorted)
    pos = jnp.zeros((TK_,), jnp.int32).at[order].set(ppos)

    nb_csum = jnp.cumsum(nbk)
    bi = jnp.arange(MAXB_, dtype=jnp.int32)
    blk_v = (bi < nb_csum[-1]).astype(jnp.int32)
    blk_e = (bi[:, None] >= nb_csum[None, :]).sum(axis=1).astype(jnp.int32)
    blk_e = jnp.where(blk_v == 1, blk_e, 0)

    xs = jnp.take(x2d, tok_padded, axis=0)
    ys = _ffn(blk_e, blk_v, xs, W1, b1, W2, b2, ws_padded[:, None])

    p = pos.reshape(T_, K_)
    out = jnp.take(ys, p[:, 0], axis=0) + jnp.take(ys, p[:, 1], axis=0)
    return out.reshape(B_, S_, D_)


# speedup vs baseline: 2.4333x; 1.6204x over previous
import functools

import jax
import jax.numpy as jnp
from jax import lax
from jax.experimental import pallas as pl
from jax.experimental.pallas import tpu as pltpu
from jax.experimental.pallas import tpu_sc as plsc

B_, S_, D_, H_, E_, K_ = 2, 2048, 1024, 4096, 8, 2
T_ = B_ * S_
TK_ = T_ * K_
BT_ = 1024
MAXB_ = TK_ // BT_ + E_
NPAD_ = MAXB_ * BT_
NH_ = 4
HT_ = H_ // NH_
TBG_ = 512
EP_ = 128
GC_ = 32
NW_ = 32
TPW_ = T_ // NW_
NCH_ = TPW_ // GC_


def _gate_body(x_ref, wg_ref, bgm_ref, i1_ref, i2_ref, w1_ref, w2_ref):
    s = jnp.dot(x_ref[...].astype(jnp.bfloat16),
                wg_ref[...].astype(jnp.bfloat16),
                preferred_element_type=jnp.float32)
    s = s + bgm_ref[...]
    iota = lax.broadcasted_iota(jnp.int32, s.shape, 1)
    big = jnp.int32(1 << 30)
    m1 = jnp.max(s, axis=1, keepdims=True)
    a1 = jnp.min(jnp.where(s >= m1, iota, big), axis=1, keepdims=True)
    s2 = jnp.where(iota == a1, -1e30, s)
    m2 = jnp.max(s2, axis=1, keepdims=True)
    a2 = jnp.min(jnp.where(s2 >= m2, iota, big), axis=1, keepdims=True)
    e2 = jnp.exp(m2 - m1)
    i1_ref[...] = a1
    i2_ref[...] = a2
    w1_ref[...] = jnp.broadcast_to(1.0 / (1.0 + e2), (s.shape[0], 128))
    w2_ref[...] = jnp.broadcast_to(e2 / (1.0 + e2), (s.shape[0], 128))


def _gate(x2d, wg_pad, bg_pad):
    return pl.pallas_call(
        _gate_body,
        grid=(T_ // TBG_,),
        in_specs=[
            pl.BlockSpec((TBG_, D_), lambda i: (i, 0)),
            pl.BlockSpec((D_, EP_), lambda i: (0, 0)),
            pl.BlockSpec((1, EP_), lambda i: (0, 0)),
        ],
        out_specs=[
            pl.BlockSpec((TBG_, 1), lambda i: (i, 0)),
            pl.BlockSpec((TBG_, 1), lambda i: (i, 0)),
            pl.BlockSpec((TBG_, 128), lambda i: (i, 0)),
            pl.BlockSpec((TBG_, 128), lambda i: (i, 0)),
        ],
        out_shape=[
            jax.ShapeDtypeStruct((T_, 1), jnp.int32),
            jax.ShapeDtypeStruct((T_, 1), jnp.int32),
            jax.ShapeDtypeStruct((T_, 128), jnp.float32),
            jax.ShapeDtypeStruct((T_, 128), jnp.float32),
        ],
    )(x2d, wg_pad, bg_pad)


def _route_body(i1_ref, i2_ref, p0_ref, p1_ref, be_ref, bv_ref, gt_ref):
    lane = lax.broadcasted_iota(jnp.int32, (128, 128), 1)
    row = lax.broadcasted_iota(jnp.int32, (128, 128), 0)
    tril = jnp.where(lane < row, 1.0, 0.0).astype(jnp.bfloat16)
    triu = jnp.where(row < lane, 1.0, 0.0).astype(jnp.bfloat16)

    for g in range(32):
        i1c = i1_ref[pl.ds(g * 128, 128), :]
        i2c = i2_ref[pl.ds(g * 128, 128), :]
        o1 = jnp.where(i1c == lane, 1.0, 0.0)
        o2 = jnp.where(i2c == lane, 1.0, 0.0)
        gt_ref[pl.ds(g, 1), :] = jnp.sum(o1, axis=0, keepdims=True)
        gt_ref[pl.ds(32 + g, 1), :] = jnp.sum(o2, axis=0, keepdims=True)

    gt0 = gt_ref[pl.ds(0, 32), :]
    gt1 = gt_ref[pl.ds(32, 32), :]
    l32 = jnp.where(lax.broadcasted_iota(jnp.int32, (32, 32), 1)
                    < lax.broadcasted_iota(jnp.int32, (32, 32), 0),
                    1.0, 0.0).astype(jnp.bfloat16)
    gt0ex = jnp.dot(l32, gt0.astype(jnp.bfloat16),
                    preferred_element_type=jnp.float32)
    gt1ex = jnp.dot(l32, gt1.astype(jnp.bfloat16),
                    preferred_element_type=jnp.float32)
    c0 = jnp.sum(gt0, axis=0, keepdims=True)
    c1 = jnp.sum(gt1, axis=0, keepdims=True)
    counts = c0 + c1
    nb = (counts.astype(jnp.int32) + BT_ - 1) // BT_
    nbf = nb.astype(jnp.bfloat16)
    ps = jnp.dot(nbf, triu, preferred_element_type=jnp.float32) * float(BT_)

    for g in range(32):
        i1c = i1_ref[pl.ds(g * 128, 128), :]
        i2c = i2_ref[pl.ds(g * 128, 128), :]
        o1 = jnp.where(i1c == lane, 1.0, 0.0)
        o2 = jnp.where(i2c == lane, 1.0, 0.0)
        loc1 = jnp.dot(tril, o1.astype(jnp.bfloat16),
                       preferred_element_type=jnp.float32)
        loc2 = jnp.dot(tril, o2.astype(jnp.bfloat16),
                       preferred_element_type=jnp.float32)
        r0 = loc1 + gt0ex[g:g + 1, :]
        r1 = loc2 + gt1ex[g:g + 1, :] + c0
        p0c = jnp.sum(o1 * (r0 + ps), axis=1, keepdims=True)
        p1c = jnp.sum(o2 * (r1 + ps), axis=1, keepdims=True)
        p0_ref[pl.ds(g * 128, 128), :] = p0c.astype(jnp.int32)
        p1_ref[pl.ds(g * 128, 128), :] = p1c.astype(jnp.int32)

    csum = ps / float(BT_) + nb.astype(jnp.float32)
    bidx = lax.broadcasted_iota(jnp.int32, (MAXB_, 128), 0)
    lane8 = lax.broadcasted_iota(jnp.int32, (MAXB_, 128), 1) < E_
    ge = jnp.where(lane8 & (bidx >= csum.astype(jnp.int32)), 1, 0)
    be = jnp.sum(ge, axis=1, keepdims=True)
    tot = jnp.sum(jnp.where(lane8, nb, 0), axis=1, keepdims=True)
    bv = jnp.where(bidx[:, :1] < tot, 1, 0)
    be_ref[...] = jnp.where(bv == 1, be, 0)
    bv_ref[...] = bv


def _route(i1, i2):
    return pl.pallas_call(
        _route_body,
        grid=(1,),
        in_specs=[
            pl.BlockSpec((T_, 1), lambda i: (0, 0)),
            pl.BlockSpec((T_, 1), lambda i: (0, 0)),
        ],
        out_specs=[
            pl.BlockSpec((T_, 1), lambda i: (0, 0)),
            pl.BlockSpec((T_, 1), lambda i: (0, 0)),
            pl.BlockSpec((MAXB_, 1), lambda i: (0, 0)),
            pl.BlockSpec((MAXB_, 1), lambda i: (0, 0)),
        ],
        out_shape=[
            jax.ShapeDtypeStruct((T_, 1), jnp.int32),
            jax.ShapeDtypeStruct((T_, 1), jnp.int32),
            jax.ShapeDtypeStruct((MAXB_, 1), jnp.int32),
            jax.ShapeDtypeStruct((MAXB_, 1), jnp.int32),
        ],
        scratch_shapes=[pltpu.VMEM((64, 128), jnp.float32)],
    )(i1, i2)


@functools.lru_cache(maxsize=None)
def _sc_kernels():
    mesh = plsc.VectorSubcoreMesh(core_axis_name="c", subcore_axis_name="s",
                                  num_cores=2, num_subcores=16)

    @functools.partial(
        pl.kernel,
        out_type=[
            jax.ShapeDtypeStruct((NPAD_, D_), jnp.float32),
            jax.ShapeDtypeStruct((NPAD_, 128), jnp.float32),
        ],
        mesh=mesh,
        scratch_types=[
            pltpu.VMEM((NCH_, GC_), jnp.int32),
            pltpu.VMEM((NCH_, GC_), jnp.int32),
            pltpu.VMEM((GC_, D_), jnp.float32),
            pltpu.VMEM((GC_, 128), jnp.float32),
            pltpu.VMEM((GC_, 128), jnp.float32),
            pltpu.SemaphoreType.DMA,
        ],
    )
    def _sc_dispatch(x_hbm, p0_hbm, p1_hbm, w1_hbm, w2_hbm, xs_hbm, ws_hbm,
                     p0_v, p1_v, xbuf, wb0, wb1, sem):
        wid = lax.axis_index("s") * 2 + lax.axis_index("c")
        tok0 = wid * TPW_
        pltpu.sync_copy(p0_hbm.at[pl.ds(wid * NCH_, NCH_)], p0_v)
        pltpu.sync_copy(p1_hbm.at[pl.ds(wid * NCH_, NCH_)], p1_v)
        for c in range(NCH_):
            pltpu.sync_copy(x_hbm.at[pl.ds(tok0 + c * GC_, GC_)], xbuf)
            pltpu.sync_copy(w1_hbm.at[pl.ds(tok0 + c * GC_, GC_)], wb0)
            pltpu.sync_copy(w2_hbm.at[pl.ds(tok0 + c * GC_, GC_)], wb1)
            a = pltpu.async_copy(xbuf, xs_hbm.at[p0_v.at[c]], sem)
            b = pltpu.async_copy(xbuf, xs_hbm.at[p1_v.at[c]], sem)
            d = pltpu.async_copy(wb0, ws_hbm.at[p0_v.at[c]], sem)
            e = pltpu.async_copy(wb1, ws_hbm.at[p1_v.at[c]], sem)
            a.wait()
            b.wait()
            d.wait()
            e.wait()

    @functools.partial(
        pl.kernel,
        out_type=jax.ShapeDtypeStruct((T_, D_), jnp.float32),
        mesh=mesh,
        scratch_types=[
            pltpu.VMEM((NCH_, GC_), jnp.int32),
            pltpu.VMEM((NCH_, GC_), jnp.int32),
            pltpu.VMEM((GC_, D_), jnp.float32),
            pltpu.VMEM((GC_, D_), jnp.float32),
            pltpu.SemaphoreType.DMA,
            pltpu.SemaphoreType.DMA,
        ],
    )
    def _sc_combine(ys_hbm, p0_hbm, p1_hbm, out_hbm, p0_v, p1_v, buf0, buf1,
                    sem0, sem1):
        wid = lax.axis_index("s") * 2 + lax.axis_index("c")
        tok0 = wid * TPW_
        pltpu.sync_copy(p0_hbm.at[pl.ds(wid * NCH_, NCH_)], p0_v)
        pltpu.sync_copy(p1_hbm.at[pl.ds(wid * NCH_, NCH_)], p1_v)
        for c in range(NCH_):
            a = pltpu.async_copy(ys_hbm.at[p0_v.at[c]], buf0, sem0)
            b = pltpu.async_copy(ys_hbm.at[p1_v.at[c]], buf1, sem1)
            a.wait()
            b.wait()

            def _row(r, carry):
                for col in range(0, D_, 16):
                    buf0[r, pl.ds(col, 16)] = (buf0[r, pl.ds(col, 16)]
                                               + buf1[r, pl.ds(col, 16)])
                return carry

            lax.fori_loop(0, GC_, _row, 0)
            pltpu.sync_copy(buf0, out_hbm.at[pl.ds(tok0 + c * GC_, GC_)])

    return _sc_dispatch, _sc_combine


def _ffn_body(be_ref, bv_ref, xs_ref, w1_ref, b1_ref, w2_ref, b2_ref,
              ws_ref, out_ref):
    h = pl.program_id(1)
    i = pl.program_id(0)

    @pl.when(bv_ref[i] == 1)
    def _():
        @pl.when(h == 0)
        def _():
            out_ref[...] = jnp.zeros_like(out_ref)

        hh = jnp.dot(xs_ref[...], w1_ref[0],
                     preferred_element_type=jnp.float32) + b1_ref[0, 0]
        hb = jnp.maximum(hh, 0.0)
        out_ref[...] += jnp.dot(hb, w2_ref[0],
                                preferred_element_type=jnp.float32)

        @pl.when(h == NH_ - 1)
        def _():
            out_ref[...] = (out_ref[...] + b2_ref[0]) * ws_ref[:, :1]


def _ffn(blk_e, blk_v, xs, W1, b1, W2, b2, ws16):
    grid_spec = pltpu.PrefetchScalarGridSpec(
        num_scalar_prefetch=2,
        grid=(MAXB_, NH_),
        in_specs=[
            pl.BlockSpec((BT_, D_), lambda i, h, be, bv: (i, 0)),
            pl.BlockSpec((1, D_, HT_),
                         lambda i, h, be, bv: (be[i], 0, h * bv[i])),
            pl.BlockSpec((1, 1, 1, HT_),
                         lambda i, h, be, bv: (be[i], h * bv[i], 0, 0)),
            pl.BlockSpec((1, HT_, D_),
                         lambda i, h, be, bv: (be[i], h * bv[i], 0)),
            pl.BlockSpec((1, 1, D_), lambda i, h, be, bv: (be[i], 0, 0)),
            pl.BlockSpec((BT_, 128), lambda i, h, be, bv: (i, 0)),
        ],
        out_specs=pl.BlockSpec((BT_, D_), lambda i, h, be, bv: (i, 0)),
    )
    return pl.pallas_call(
        _ffn_body,
        grid_spec=grid_spec,
        out_shape=jax.ShapeDtypeStruct((NPAD_, D_), jnp.float32),
    )(blk_e, blk_v, xs, W1, b1.reshape(E_, NH_, 1, HT_), W2,
      b2.reshape(E_, 1, D_), ws16)


def kernel(x, Wg, bg, W1, b1, W2, b2):
    x2d = x.reshape(T_, D_)
    wg_pad = jnp.zeros((D_, EP_), jnp.float32).at[:, :E_].set(Wg)
    bg_pad = jnp.full((1, EP_), -1e30, jnp.float32).at[0, :E_].set(bg)
    i1, i2, w1, w2 = _gate(x2d, wg_pad, bg_pad)
    p0, p1, blk_e, blk_v = _route(i1, i2)

    dispatch, combine = _sc_kernels()
    p0r = p0.reshape(T_ // GC_, GC_)
    p1r = p1.reshape(T_ // GC_, GC_)
    xs, ws16 = dispatch(x2d, p0r, p1r, w1, w2)
    ys = _ffn(blk_e.reshape(MAXB_), blk_v.reshape(MAXB_), xs, W1, b1, W2,
              b2, ws16)
    out = combine(ys, p0r, p1r)
    return out.reshape(B_, S_, D_)


# speedup vs baseline: 2.5602x; 1.0522x over previous
import functools

import jax
import jax.numpy as jnp
from jax import lax
from jax.experimental import pallas as pl
from jax.experimental.pallas import tpu as pltpu
from jax.experimental.pallas import tpu_sc as plsc

B_, S_, D_, H_, E_, K_ = 2, 2048, 1024, 4096, 8, 2
T_ = B_ * S_
TK_ = T_ * K_
BT_ = 1024
MAXB_ = TK_ // BT_ + E_
NPAD_ = MAXB_ * BT_
NH_ = 4
HT_ = H_ // NH_
TBG_ = 512
EP_ = 128
GC_ = 16
DP_ = D_ // 2
NW_ = 32
TPW_ = T_ // NW_
NCH_ = TPW_ // GC_


def _gate_body(x_ref, wg_ref, bgm_ref, i1_ref, i2_ref, w1_ref, w2_ref):
    s = jnp.dot(x_ref[...].astype(jnp.bfloat16),
                wg_ref[...].astype(jnp.bfloat16),
                preferred_element_type=jnp.float32)
    s = s + bgm_ref[...]
    iota = lax.broadcasted_iota(jnp.int32, s.shape, 1)
    big = jnp.int32(1 << 30)
    m1 = jnp.max(s, axis=1, keepdims=True)
    a1 = jnp.min(jnp.where(s >= m1, iota, big), axis=1, keepdims=True)
    s2 = jnp.where(iota == a1, -1e30, s)
    m2 = jnp.max(s2, axis=1, keepdims=True)
    a2 = jnp.min(jnp.where(s2 >= m2, iota, big), axis=1, keepdims=True)
    e2 = jnp.exp(m2 - m1)
    i1_ref[...] = a1
    i2_ref[...] = a2
    w1_ref[...] = jnp.broadcast_to(1.0 / (1.0 + e2), (s.shape[0], 128))
    w2_ref[...] = jnp.broadcast_to(e2 / (1.0 + e2), (s.shape[0], 128))


def _gate(x2d, wg_pad, bg_pad):
    return pl.pallas_call(
        _gate_body,
        grid=(T_ // TBG_,),
        in_specs=[
            pl.BlockSpec((TBG_, D_), lambda i: (i, 0)),
            pl.BlockSpec((D_, EP_), lambda i: (0, 0)),
            pl.BlockSpec((1, EP_), lambda i: (0, 0)),
        ],
        out_specs=[
            pl.BlockSpec((TBG_, 1), lambda i: (i, 0)),
            pl.BlockSpec((TBG_, 1), lambda i: (i, 0)),
            pl.BlockSpec((TBG_, 128), lambda i: (i, 0)),
            pl.BlockSpec((TBG_, 128), lambda i: (i, 0)),
        ],
        out_shape=[
            jax.ShapeDtypeStruct((T_, 1), jnp.int32),
            jax.ShapeDtypeStruct((T_, 1), jnp.int32),
            jax.ShapeDtypeStruct((T_, 128), jnp.float32),
            jax.ShapeDtypeStruct((T_, 128), jnp.float32),
        ],
    )(x2d, wg_pad, bg_pad)


def _route_body(i1_ref, i2_ref, p0_ref, p1_ref, be_ref, bv_ref, gt_ref):
    lane = lax.broadcasted_iota(jnp.int32, (128, 128), 1)
    row = lax.broadcasted_iota(jnp.int32, (128, 128), 0)
    tril = jnp.where(lane < row, 1.0, 0.0).astype(jnp.bfloat16)
    triu = jnp.where(row < lane, 1.0, 0.0).astype(jnp.bfloat16)

    for g in range(32):
        i1c = i1_ref[pl.ds(g * 128, 128), :]
        i2c = i2_ref[pl.ds(g * 128, 128), :]
        o1 = jnp.where(i1c == lane, 1.0, 0.0)
        o2 = jnp.where(i2c == lane, 1.0, 0.0)
        gt_ref[pl.ds(g, 1), :] = jnp.sum(o1, axis=0, keepdims=True)
        gt_ref[pl.ds(32 + g, 1), :] = jnp.sum(o2, axis=0, keepdims=True)

    gt0 = gt_ref[pl.ds(0, 32), :]
    gt1 = gt_ref[pl.ds(32, 32), :]
    l32 = jnp.where(lax.broadcasted_iota(jnp.int32, (32, 32), 1)
                    < lax.broadcasted_iota(jnp.int32, (32, 32), 0),
                    1.0, 0.0).astype(jnp.bfloat16)
    gt0ex = jnp.dot(l32, gt0.astype(jnp.bfloat16),
                    preferred_element_type=jnp.float32)
    gt1ex = jnp.dot(l32, gt1.astype(jnp.bfloat16),
                    preferred_element_type=jnp.float32)
    c0 = jnp.sum(gt0, axis=0, keepdims=True)
    c1 = jnp.sum(gt1, axis=0, keepdims=True)
    counts = c0 + c1
    nb = (counts.astype(jnp.int32) + BT_ - 1) // BT_
    nbf = nb.astype(jnp.bfloat16)
    ps = jnp.dot(nbf, triu, preferred_element_type=jnp.float32) * float(BT_)

    for g in range(32):
        i1c = i1_ref[pl.ds(g * 128, 128), :]
        i2c = i2_ref[pl.ds(g * 128, 128), :]
        o1 = jnp.where(i1c == lane, 1.0, 0.0)
        o2 = jnp.where(i2c == lane, 1.0, 0.0)
        loc1 = jnp.dot(tril, o1.astype(jnp.bfloat16),
                       preferred_element_type=jnp.float32)
        loc2 = jnp.dot(tril, o2.astype(jnp.bfloat16),
                       preferred_element_type=jnp.float32)
        r0 = loc1 + gt0ex[g:g + 1, :]
        r1 = loc2 + gt1ex[g:g + 1, :] + c0
        p0c = jnp.sum(o1 * (r0 + ps), axis=1, keepdims=True)
        p1c = jnp.sum(o2 * (r1 + ps), axis=1, keepdims=True)
        p0_ref[pl.ds(g * 128, 128), :] = p0c.astype(jnp.int32)
        p1_ref[pl.ds(g * 128, 128), :] = p1c.astype(jnp.int32)

    csum = ps / float(BT_) + nb.astype(jnp.float32)
    bidx = lax.broadcasted_iota(jnp.int32, (MAXB_, 128), 0)
    lane8 = lax.broadcasted_iota(jnp.int32, (MAXB_, 128), 1) < E_
    ge = jnp.where(lane8 & (bidx >= csum.astype(jnp.int32)), 1, 0)
    be = jnp.sum(ge, axis=1, keepdims=True)
    tot = jnp.sum(jnp.where(lane8, nb, 0), axis=1, keepdims=True)
    bv = jnp.where(bidx[:, :1] < tot, 1, 0)
    be_ref[...] = jnp.where(bv == 1, be, 0)
    bv_ref[...] = bv


def _route(i1, i2):
    return pl.pallas_call(
        _route_body,
        grid=(1,),
        in_specs=[
            pl.BlockSpec((T_, 1), lambda i: (0, 0)),
            pl.BlockSpec((T_, 1), lambda i: (0, 0)),
        ],
        out_specs=[
            pl.BlockSpec((T_, 1), lambda i: (0, 0)),
            pl.BlockSpec((T_, 1), lambda i: (0, 0)),
            pl.BlockSpec((MAXB_, 1), lambda i: (0, 0)),
            pl.BlockSpec((MAXB_, 1), lambda i: (0, 0)),
        ],
        out_shape=[
            jax.ShapeDtypeStruct((T_, 1), jnp.int32),
            jax.ShapeDtypeStruct((T_, 1), jnp.int32),
            jax.ShapeDtypeStruct((MAXB_, 1), jnp.int32),
            jax.ShapeDtypeStruct((MAXB_, 1), jnp.int32),
        ],
        scratch_shapes=[pltpu.VMEM((64, 128), jnp.float32)],
    )(i1, i2)


@functools.lru_cache(maxsize=None)
def _sc_kernels():
    mesh = plsc.VectorSubcoreMesh(core_axis_name="c", subcore_axis_name="s",
                                  num_cores=2, num_subcores=16)

    @functools.partial(
        pl.kernel,
        out_type=[
            jax.ShapeDtypeStruct((NPAD_, D_), jnp.float32),
            jax.ShapeDtypeStruct((NPAD_, 128), jnp.float32),
        ],
        mesh=mesh,
        scratch_types=[
            pltpu.VMEM((NCH_, GC_), jnp.int32),
            pltpu.VMEM((NCH_, GC_), jnp.int32),
            pltpu.VMEM((GC_, D_), jnp.float32),
            pltpu.VMEM((GC_, D_), jnp.float32),
            pltpu.VMEM((GC_, 128), jnp.float32),
            pltpu.VMEM((GC_, 128), jnp.float32),
            pltpu.VMEM((GC_, 128), jnp.float32),
            pltpu.VMEM((GC_, 128), jnp.float32),
            pltpu.SemaphoreType.DMA,
            pltpu.SemaphoreType.DMA,
            pltpu.SemaphoreType.DMA,
            pltpu.SemaphoreType.DMA,
        ],
    )
    def _sc_dispatch(x_hbm, p0_hbm, p1_hbm, w1_hbm, w2_hbm, xs_hbm, ws_hbm,
                     p0_v, p1_v, xb0, xb1, wa0, wa1, wb0, wb1,
                     si0, si1, so0, so1):
        wid = lax.axis_index("s") * 2 + lax.axis_index("c")
        tok0 = wid * TPW_
        pltpu.sync_copy(p0_hbm.at[pl.ds(wid * NCH_, NCH_)], p0_v)
        pltpu.sync_copy(p1_hbm.at[pl.ds(wid * NCH_, NCH_)], p1_v)
        xb = (xb0, xb1)
        wa = (wa0, wa1)
        wb = (wb0, wb1)
        si = (si0, si1)
        so = (so0, so1)

        def start_in(c, b):
            sl = pl.ds(tok0 + c * GC_, GC_)
            return (pltpu.async_copy(x_hbm.at[sl], xb[b], si[b]),
                    pltpu.async_copy(w1_hbm.at[sl], wa[b], si[b]),
                    pltpu.async_copy(w2_hbm.at[sl], wb[b], si[b]))

        pend_in = start_in(0, 0)
        pend_sc = [None, None]
        for c in range(NCH_):
            b = c % 2
            for h in pend_in:
                h.wait()
            if c + 1 < NCH_:
                if pend_sc[1 - b] is not None:
                    for h in pend_sc[1 - b]:
                        h.wait()
                    pend_sc[1 - b] = None
                pend_in = start_in(c + 1, 1 - b)
            pend_sc[b] = (
                pltpu.async_copy(xb[b], xs_hbm.at[p0_v.at[c]], so[b]),
                pltpu.async_copy(xb[b], xs_hbm.at[p1_v.at[c]], so[b]),
                pltpu.async_copy(wa[b], ws_hbm.at[p0_v.at[c]], so[b]),
                pltpu.async_copy(wb[b], ws_hbm.at[p1_v.at[c]], so[b]),
            )
        for bb in (0, 1):
            if pend_sc[bb] is not None:
                for h in pend_sc[bb]:
                    h.wait()

    @functools.partial(
        pl.kernel,
        out_type=jax.ShapeDtypeStruct((T_, D_), jnp.float32),
        mesh=mesh,
        scratch_types=[
            pltpu.VMEM((NCH_, GC_), jnp.int32),
            pltpu.VMEM((NCH_, GC_), jnp.int32),
            pltpu.VMEM((GC_, D_), jnp.float32),
            pltpu.VMEM((GC_, D_), jnp.float32),
            pltpu.VMEM((GC_, D_), jnp.float32),
            pltpu.VMEM((GC_, D_), jnp.float32),
            pltpu.SemaphoreType.DMA,
            pltpu.SemaphoreType.DMA,
            pltpu.SemaphoreType.DMA,
            pltpu.SemaphoreType.DMA,
        ],
    )
    def _sc_combine(ys_hbm, p0_hbm, p1_hbm, out_hbm, p0_v, p1_v,
                    a0, a1, b0, b1, sg0, sg1, so0, so1):
        wid = lax.axis_index("s") * 2 + lax.axis_index("c")
        tok0 = wid * TPW_
        pltpu.sync_copy(p0_hbm.at[pl.ds(wid * NCH_, NCH_)], p0_v)
        pltpu.sync_copy(p1_hbm.at[pl.ds(wid * NCH_, NCH_)], p1_v)
        ab = (a0, a1)
        bb_ = (b0, b1)
        sg = (sg0, sg1)
        so = (so0, so1)

        def start_g(c, b):
            return (pltpu.async_copy(ys_hbm.at[p0_v.at[c]], ab[b], sg[b]),
                    pltpu.async_copy(ys_hbm.at[p1_v.at[c]], bb_[b], sg[b]))

        pend_g = start_g(0, 0)
        pend_o = [None, None]
        for c in range(NCH_):
            b = c % 2
            for h in pend_g:
                h.wait()
            if c + 1 < NCH_:
                if pend_o[1 - b] is not None:
                    pend_o[1 - b].wait()
                    pend_o[1 - b] = None
                pend_g = start_g(c + 1, 1 - b)

            def _row(r, carry, _ba=ab[b], _bb=bb_[b]):
                for col in range(0, D_, 16):
                    _ba[r, pl.ds(col, 16)] = (_ba[r, pl.ds(col, 16)]
                                              + _bb[r, pl.ds(col, 16)])
                return carry

            lax.fori_loop(0, GC_, _row, 0)
            pend_o[b] = pltpu.async_copy(
                ab[b], out_hbm.at[pl.ds(tok0 + c * GC_, GC_)], so[b])
        for z in (0, 1):
            if pend_o[z] is not None:
                pend_o[z].wait()

    return _sc_dispatch, _sc_combine


def _ffn_body(be_ref, bv_ref, xs_ref, w1_ref, b1_ref, w2_ref, b2_ref,
              ws_ref, out_ref):
    h = pl.program_id(1)
    i = pl.program_id(0)

    @pl.when(bv_ref[i] == 1)
    def _():
        @pl.when(h == 0)
        def _():
            out_ref[...] = jnp.zeros_like(out_ref)

        hh = jnp.dot(xs_ref[...].astype(jnp.bfloat16),
                     w1_ref[0].astype(jnp.bfloat16),
                     preferred_element_type=jnp.float32) + b1_ref[0, 0]
        hb = jnp.maximum(hh, 0.0).astype(jnp.bfloat16)
        out_ref[...] += jnp.dot(hb, w2_ref[0].astype(jnp.bfloat16),
                                preferred_element_type=jnp.float32)

        @pl.when(h == NH_ - 1)
        def _():
            out_ref[...] = (out_ref[...] + b2_ref[0]) * ws_ref[:, :1]


def _ffn(blk_e, blk_v, xs, W1, b1, W2, b2, ws16):
    grid_spec = pltpu.PrefetchScalarGridSpec(
        num_scalar_prefetch=2,
        grid=(MAXB_, NH_),
        in_specs=[
            pl.BlockSpec((BT_, D_), lambda i, h, be, bv: (i, 0)),
            pl.BlockSpec((1, D_, HT_),
                         lambda i, h, be, bv: (be[i], 0, h * bv[i])),
            pl.BlockSpec((1, 1, 1, HT_),
                         lambda i, h, be, bv: (be[i], h * bv[i], 0, 0)),
            pl.BlockSpec((1, HT_, D_),
                         lambda i, h, be, bv: (be[i], h * bv[i], 0)),
            pl.BlockSpec((1, 1, D_), lambda i, h, be, bv: (be[i], 0, 0)),
            pl.BlockSpec((BT_, 128), lambda i, h, be, bv: (i, 0)),
        ],
        out_specs=pl.BlockSpec((BT_, D_), lambda i, h, be, bv: (i, 0)),
    )
    return pl.pallas_call(
        _ffn_body,
        grid_spec=grid_spec,
        out_shape=jax.ShapeDtypeStruct((NPAD_, D_), jnp.float32),
    )(blk_e, blk_v, xs, W1, b1.reshape(E_, NH_, 1, HT_), W2,
      b2.reshape(E_, 1, D_), ws16)


def kernel(x, Wg, bg, W1, b1, W2, b2):
    x2d = x.reshape(T_, D_)
    wg_pad = jnp.zeros((D_, EP_), jnp.float32).at[:, :E_].set(Wg)
    bg_pad = jnp.full((1, EP_), -1e30, jnp.float32).at[0, :E_].set(bg)
    i1, i2, w1, w2 = _gate(x2d, wg_pad, bg_pad)
    p0, p1, blk_e, blk_v = _route(i1, i2)

    dispatch, combine = _sc_kernels()
    p0r = p0.reshape(T_ // GC_, GC_)
    p1r = p1.reshape(T_ // GC_, GC_)
    xs, ws16 = dispatch(x2d, p0r, p1r, w1, w2)
    ys = _ffn(blk_e.reshape(MAXB_), blk_v.reshape(MAXB_), xs, W1, b1, W2,
              b2, ws16)
    out = combine(ys, p0r, p1r)
    return out.reshape(B_, S_, D_)


# speedup vs baseline: 2.6086x; 1.0189x over previous
import functools

import jax
import jax.numpy as jnp
from jax import lax
from jax.experimental import pallas as pl
from jax.experimental.pallas import tpu as pltpu
from jax.experimental.pallas import tpu_sc as plsc

B_, S_, D_, H_, E_, K_ = 2, 2048, 1024, 4096, 8, 2
T_ = B_ * S_
TK_ = T_ * K_
BT_ = 1024
MAXB_ = TK_ // BT_ + E_
NPAD_ = MAXB_ * BT_
NH_ = 2
HT_ = H_ // NH_
TBG_ = 512
EP_ = 128
GC_ = 16
DP_ = D_ // 2
NW_ = 32
TPW_ = T_ // NW_
NCH_ = TPW_ // GC_


def _gate_body(x_ref, wg_ref, bgm_ref, i1_ref, i2_ref, w1_ref, w2_ref):
    s = jnp.dot(x_ref[...].astype(jnp.bfloat16),
                wg_ref[...].astype(jnp.bfloat16),
                preferred_element_type=jnp.float32)
    s = s + bgm_ref[...]
    iota = lax.broadcasted_iota(jnp.int32, s.shape, 1)
    big = jnp.int32(1 << 30)
    m1 = jnp.max(s, axis=1, keepdims=True)
    a1 = jnp.min(jnp.where(s >= m1, iota, big), axis=1, keepdims=True)
    s2 = jnp.where(iota == a1, -1e30, s)
    m2 = jnp.max(s2, axis=1, keepdims=True)
    a2 = jnp.min(jnp.where(s2 >= m2, iota, big), axis=1, keepdims=True)
    e2 = jnp.exp(m2 - m1)
    i1_ref[...] = a1
    i2_ref[...] = a2
    w1_ref[...] = jnp.broadcast_to(1.0 / (1.0 + e2), (s.shape[0], 128))
    w2_ref[...] = jnp.broadcast_to(e2 / (1.0 + e2), (s.shape[0], 128))


def _gate(x2d, wg_pad, bg_pad):
    return pl.pallas_call(
        _gate_body,
        grid=(T_ // TBG_,),
        in_specs=[
            pl.BlockSpec((TBG_, D_), lambda i: (i, 0)),
            pl.BlockSpec((D_, EP_), lambda i: (0, 0)),
            pl.BlockSpec((1, EP_), lambda i: (0, 0)),
        ],
        out_specs=[
            pl.BlockSpec((TBG_, 1), lambda i: (i, 0)),
            pl.BlockSpec((TBG_, 1), lambda i: (i, 0)),
            pl.BlockSpec((TBG_, 128), lambda i: (i, 0)),
            pl.BlockSpec((TBG_, 128), lambda i: (i, 0)),
        ],
        out_shape=[
            jax.ShapeDtypeStruct((T_, 1), jnp.int32),
            jax.ShapeDtypeStruct((T_, 1), jnp.int32),
            jax.ShapeDtypeStruct((T_, 128), jnp.float32),
            jax.ShapeDtypeStruct((T_, 128), jnp.float32),
        ],
    )(x2d, wg_pad, bg_pad)


def _route_body(i1_ref, i2_ref, p0_ref, p1_ref, be_ref, bv_ref, gt_ref):
    lane = lax.broadcasted_iota(jnp.int32, (128, 128), 1)
    row = lax.broadcasted_iota(jnp.int32, (128, 128), 0)
    tril = jnp.where(lane < row, 1.0, 0.0).astype(jnp.bfloat16)
    triu = jnp.where(row < lane, 1.0, 0.0).astype(jnp.bfloat16)

    for g in range(32):
        i1c = i1_ref[pl.ds(g * 128, 128), :]
        i2c = i2_ref[pl.ds(g * 128, 128), :]
        o1 = jnp.where(i1c == lane, 1.0, 0.0)
        o2 = jnp.where(i2c == lane, 1.0, 0.0)
        gt_ref[pl.ds(g, 1), :] = jnp.sum(o1, axis=0, keepdims=True)
        gt_ref[pl.ds(32 + g, 1), :] = jnp.sum(o2, axis=0, keepdims=True)

    gt0 = gt_ref[pl.ds(0, 32), :]
    gt1 = gt_ref[pl.ds(32, 32), :]
    l32 = jnp.where(lax.broadcasted_iota(jnp.int32, (32, 32), 1)
                    < lax.broadcasted_iota(jnp.int32, (32, 32), 0),
                    1.0, 0.0).astype(jnp.bfloat16)
    gt0ex = jnp.dot(l32, gt0.astype(jnp.bfloat16),
                    preferred_element_type=jnp.float32)
    gt1ex = jnp.dot(l32, gt1.astype(jnp.bfloat16),
                    preferred_element_type=jnp.float32)
    c0 = jnp.sum(gt0, axis=0, keepdims=True)
    c1 = jnp.sum(gt1, axis=0, keepdims=True)
    counts = c0 + c1
    nb = (counts.astype(jnp.int32) + BT_ - 1) // BT_
    nbf = nb.astype(jnp.bfloat16)
    ps = jnp.dot(nbf, triu, preferred_element_type=jnp.float32) * float(BT_)

    for g in range(32):
        i1c = i1_ref[pl.ds(g * 128, 128), :]
        i2c = i2_ref[pl.ds(g * 128, 128), :]
        o1 = jnp.where(i1c == lane, 1.0, 0.0)
        o2 = jnp.where(i2c == lane, 1.0, 0.0)
        loc1 = jnp.dot(tril, o1.astype(jnp.bfloat16),
                       preferred_element_type=jnp.float32)
        loc2 = jnp.dot(tril, o2.astype(jnp.bfloat16),
                       preferred_element_type=jnp.float32)
        r0 = loc1 + gt0ex[g:g + 1, :]
        r1 = loc2 + gt1ex[g:g + 1, :] + c0
        p0c = jnp.sum(o1 * (r0 + ps), axis=1, keepdims=True)
        p1c = jnp.sum(o2 * (r1 + ps), axis=1, keepdims=True)
        p0_ref[pl.ds(g * 128, 128), :] = p0c.astype(jnp.int32)
        p1_ref[pl.ds(g * 128, 128), :] = p1c.astype(jnp.int32)

    csum = ps / float(BT_) + nb.astype(jnp.float32)
    bidx = lax.broadcasted_iota(jnp.int32, (MAXB_, 128), 0)
    lane8 = lax.broadcasted_iota(jnp.int32, (MAXB_, 128), 1) < E_
    ge = jnp.where(lane8 & (bidx >= csum.astype(jnp.int32)), 1, 0)
    be = jnp.sum(ge, axis=1, keepdims=True)
    tot = jnp.sum(jnp.where(lane8, nb, 0), axis=1, keepdims=True)
    bv = jnp.where(bidx[:, :1] < tot, 1, 0)
    be_ref[...] = jnp.where(bv == 1, be, 0)
    bv_ref[...] = bv


def _route(i1, i2):
    return pl.pallas_call(
        _route_body,
        grid=(1,),
        in_specs=[
            pl.BlockSpec((T_, 1), lambda i: (0, 0)),
            pl.BlockSpec((T_, 1), lambda i: (0, 0)),
        ],
        out_specs=[
            pl.BlockSpec((T_, 1), lambda i: (0, 0)),
            pl.BlockSpec((T_, 1), lambda i: (0, 0)),
            pl.BlockSpec((MAXB_, 1), lambda i: (0, 0)),
            pl.BlockSpec((MAXB_, 1), lambda i: (0, 0)),
        ],
        out_shape=[
            jax.ShapeDtypeStruct((T_, 1), jnp.int32),
            jax.ShapeDtypeStruct((T_, 1), jnp.int32),
            jax.ShapeDtypeStruct((MAXB_, 1), jnp.int32),
            jax.ShapeDtypeStruct((MAXB_, 1), jnp.int32),
        ],
        scratch_shapes=[pltpu.VMEM((64, 128), jnp.float32)],
    )(i1, i2)


@functools.lru_cache(maxsize=None)
def _sc_kernels():
    mesh = plsc.VectorSubcoreMesh(core_axis_name="c", subcore_axis_name="s",
                                  num_cores=2, num_subcores=16)

    @functools.partial(
        pl.kernel,
        out_type=[
            jax.ShapeDtypeStruct((NPAD_, D_), jnp.float32),
            jax.ShapeDtypeStruct((NPAD_, 128), jnp.float32),
        ],
        mesh=mesh,
        scratch_types=[
            pltpu.VMEM((NCH_, GC_), jnp.int32),
            pltpu.VMEM((NCH_, GC_), jnp.int32),
            pltpu.VMEM((GC_, D_), jnp.float32),
            pltpu.VMEM((GC_, D_), jnp.float32),
            pltpu.VMEM((GC_, 128), jnp.float32),
            pltpu.VMEM((GC_, 128), jnp.float32),
            pltpu.VMEM((GC_, 128), jnp.float32),
            pltpu.VMEM((GC_, 128), jnp.float32),
            pltpu.SemaphoreType.DMA,
            pltpu.SemaphoreType.DMA,
            pltpu.SemaphoreType.DMA,
            pltpu.SemaphoreType.DMA,
        ],
    )
    def _sc_dispatch(x_hbm, p0_hbm, p1_hbm, w1_hbm, w2_hbm, xs_hbm, ws_hbm,
                     p0_v, p1_v, xb0, xb1, wa0, wa1, wb0, wb1,
                     si0, si1, so0, so1):
        wid = lax.axis_index("s") * 2 + lax.axis_index("c")
        tok0 = wid * TPW_
        pltpu.sync_copy(p0_hbm.at[pl.ds(wid * NCH_, NCH_)], p0_v)
        pltpu.sync_copy(p1_hbm.at[pl.ds(wid * NCH_, NCH_)], p1_v)
        xb = (xb0, xb1)
        wa = (wa0, wa1)
        wb = (wb0, wb1)
        si = (si0, si1)
        so = (so0, so1)

        def start_in(c, b):
            sl = pl.ds(tok0 + c * GC_, GC_)
            return (pltpu.async_copy(x_hbm.at[sl], xb[b], si[b]),
                    pltpu.async_copy(w1_hbm.at[sl], wa[b], si[b]),
                    pltpu.async_copy(w2_hbm.at[sl], wb[b], si[b]))

        pend_in = start_in(0, 0)
        pend_sc = [None, None]
        for c in range(NCH_):
            b = c % 2
            for h in pend_in:
                h.wait()
            if c + 1 < NCH_:
                if pend_sc[1 - b] is not None:
                    for h in pend_sc[1 - b]:
                        h.wait()
                    pend_sc[1 - b] = None
                pend_in = start_in(c + 1, 1 - b)
            pend_sc[b] = (
                pltpu.async_copy(xb[b], xs_hbm.at[p0_v.at[c]], so[b]),
                pltpu.async_copy(xb[b], xs_hbm.at[p1_v.at[c]], so[b]),
                pltpu.async_copy(wa[b], ws_hbm.at[p0_v.at[c]], so[b]),
                pltpu.async_copy(wb[b], ws_hbm.at[p1_v.at[c]], so[b]),
            )
        for bb in (0, 1):
            if pend_sc[bb] is not None:
                for h in pend_sc[bb]:
                    h.wait()

    @functools.partial(
        pl.kernel,
        out_type=jax.ShapeDtypeStruct((T_, D_), jnp.float32),
        mesh=mesh,
        scratch_types=[
            pltpu.VMEM((NCH_, GC_), jnp.int32),
            pltpu.VMEM((NCH_, GC_), jnp.int32),
            pltpu.VMEM((GC_, D_), jnp.float32),
            pltpu.VMEM((GC_, D_), jnp.float32),
            pltpu.VMEM((GC_, D_), jnp.float32),
            pltpu.VMEM((GC_, D_), jnp.float32),
            pltpu.SemaphoreType.DMA,
            pltpu.SemaphoreType.DMA,
            pltpu.SemaphoreType.DMA,
            pltpu.SemaphoreType.DMA,
        ],
    )
    def _sc_combine(ys_hbm, p0_hbm, p1_hbm, out_hbm, p0_v, p1_v,
                    a0, a1, b0, b1, sg0, sg1, so0, so1):
        wid = lax.axis_index("s") * 2 + lax.axis_index("c")
        tok0 = wid * TPW_
        pltpu.sync_copy(p0_hbm.at[pl.ds(wid * NCH_, NCH_)], p0_v)
        pltpu.sync_copy(p1_hbm.at[pl.ds(wid * NCH_, NCH_)], p1_v)
        ab = (a0, a1)
        bb_ = (b0, b1)
        sg = (sg0, sg1)
        so = (so0, so1)

        def start_g(c, b):
            return (pltpu.async_copy(ys_hbm.at[p0_v.at[c]], ab[b], sg[b]),
                    pltpu.async_copy(ys_hbm.at[p1_v.at[c]], bb_[b], sg[b]))

        pend_g = start_g(0, 0)
        pend_o = [None, None]
        for c in range(NCH_):
            b = c % 2
            for h in pend_g:
                h.wait()
            if c + 1 < NCH_:
                if pend_o[1 - b] is not None:
                    pend_o[1 - b].wait()
                    pend_o[1 - b] = None
                pend_g = start_g(c + 1, 1 - b)

            def _row(r, carry, _ba=ab[b], _bb=bb_[b]):
                for col in range(0, D_, 16):
                    _ba[r, pl.ds(col, 16)] = (_ba[r, pl.ds(col, 16)]
                                              + _bb[r, pl.ds(col, 16)])
                return carry

            lax.fori_loop(0, GC_, _row, 0)
            pend_o[b] = pltpu.async_copy(
                ab[b], out_hbm.at[pl.ds(tok0 + c * GC_, GC_)], so[b])
        for z in (0, 1):
            if pend_o[z] is not None:
                pend_o[z].wait()

    return _sc_dispatch, _sc_combine


def _ffn_body(be_ref, bv_ref, xs_ref, w1_ref, b1_ref, w2_ref, b2_ref,
              ws_ref, out_ref):
    h = pl.program_id(1)
    i = pl.program_id(0)

    @pl.when(bv_ref[i] == 1)
    def _():
        @pl.when(h == 0)
        def _():
            out_ref[...] = jnp.zeros_like(out_ref)

        hh = jnp.dot(xs_ref[...].astype(jnp.bfloat16),
                     w1_ref[0].astype(jnp.bfloat16),
                     preferred_element_type=jnp.float32) + b1_ref[0, 0]
        hb = jnp.maximum(hh, 0.0).astype(jnp.bfloat16)
        out_ref[...] += jnp.dot(hb, w2_ref[0].astype(jnp.bfloat16),
                                preferred_element_type=jnp.float32)

        @pl.when(h == NH_ - 1)
        def _():
            out_ref[...] = (out_ref[...] + b2_ref[0]) * ws_ref[:, :1]


def _ffn(blk_e, blk_v, xs, W1, b1, W2, b2, ws16):
    grid_spec = pltpu.PrefetchScalarGridSpec(
        num_scalar_prefetch=2,
        grid=(MAXB_, NH_),
        in_specs=[
            pl.BlockSpec((BT_, D_), lambda i, h, be, bv: (i, 0)),
            pl.BlockSpec((1, D_, HT_),
                         lambda i, h, be, bv: (be[i], 0, h * bv[i])),
            pl.BlockSpec((1, 1, 1, HT_),
                         lambda i, h, be, bv: (be[i], h * bv[i], 0, 0)),
            pl.BlockSpec((1, HT_, D_),
                         lambda i, h, be, bv: (be[i], h * bv[i], 0)),
            pl.BlockSpec((1, 1, D_), lambda i, h, be, bv: (be[i], 0, 0)),
            pl.BlockSpec((BT_, 128), lambda i, h, be, bv: (i, 0)),
        ],
        out_specs=pl.BlockSpec((BT_, D_), lambda i, h, be, bv: (i, 0)),
    )
    return pl.pallas_call(
        _ffn_body,
        grid_spec=grid_spec,
        out_shape=jax.ShapeDtypeStruct((NPAD_, D_), jnp.float32),
    )(blk_e, blk_v, xs, W1, b1.reshape(E_, NH_, 1, HT_), W2,
      b2.reshape(E_, 1, D_), ws16)


def kernel(x, Wg, bg, W1, b1, W2, b2):
    x2d = x.reshape(T_, D_)
    wg_pad = jnp.zeros((D_, EP_), jnp.float32).at[:, :E_].set(Wg)
    bg_pad = jnp.full((1, EP_), -1e30, jnp.float32).at[0, :E_].set(bg)
    i1, i2, w1, w2 = _gate(x2d, wg_pad, bg_pad)
    p0, p1, blk_e, blk_v = _route(i1, i2)

    dispatch, combine = _sc_kernels()
    p0r = p0.reshape(T_ // GC_, GC_)
    p1r = p1.reshape(T_ // GC_, GC_)
    xs, ws16 = dispatch(x2d, p0r, p1r, w1, w2)
    ys = _ffn(blk_e.reshape(MAXB_), blk_v.reshape(MAXB_), xs, W1, b1, W2,
              b2, ws16)
    out = combine(ys, p0r, p1r)
    return out.reshape(B_, S_, D_)


# speedup vs baseline: 2.6220x; 1.0051x over previous
import functools

import jax
import jax.numpy as jnp
from jax import lax
from jax.experimental import pallas as pl
from jax.experimental.pallas import tpu as pltpu
from jax.experimental.pallas import tpu_sc as plsc

B_, S_, D_, H_, E_, K_ = 2, 2048, 1024, 4096, 8, 2
T_ = B_ * S_
TK_ = T_ * K_
BT_ = 1024
MAXB_ = TK_ // BT_ + E_
NPAD_ = MAXB_ * BT_
NH_ = 2
HT_ = H_ // NH_
TBG_ = 512
EP_ = 128
GC_ = 16
GD_ = 32
NCD_ = 128 // GD_
DP_ = D_ // 2
NW_ = 32
TPW_ = T_ // NW_
NCH_ = TPW_ // GC_


def _gate_body(x_ref, wg_ref, bgm_ref, i1_ref, i2_ref, w1_ref, w2_ref):
    s = jnp.dot(x_ref[...].astype(jnp.bfloat16),
                wg_ref[...].astype(jnp.bfloat16),
                preferred_element_type=jnp.float32)
    s = s + bgm_ref[...]
    iota = lax.broadcasted_iota(jnp.int32, s.shape, 1)
    big = jnp.int32(1 << 30)
    m1 = jnp.max(s, axis=1, keepdims=True)
    a1 = jnp.min(jnp.where(s >= m1, iota, big), axis=1, keepdims=True)
    s2 = jnp.where(iota == a1, -1e30, s)
    m2 = jnp.max(s2, axis=1, keepdims=True)
    a2 = jnp.min(jnp.where(s2 >= m2, iota, big), axis=1, keepdims=True)
    e2 = jnp.exp(m2 - m1)
    i1_ref[...] = a1
    i2_ref[...] = a2
    w1_ref[...] = jnp.broadcast_to(1.0 / (1.0 + e2), (s.shape[0], 128))
    w2_ref[...] = jnp.broadcast_to(e2 / (1.0 + e2), (s.shape[0], 128))


def _gate(x2d, wg_pad, bg_pad):
    return pl.pallas_call(
        _gate_body,
        grid=(T_ // TBG_,),
        in_specs=[
            pl.BlockSpec((TBG_, D_), lambda i: (i, 0)),
            pl.BlockSpec((D_, EP_), lambda i: (0, 0)),
            pl.BlockSpec((1, EP_), lambda i: (0, 0)),
        ],
        out_specs=[
            pl.BlockSpec((TBG_, 1), lambda i: (i, 0)),
            pl.BlockSpec((TBG_, 1), lambda i: (i, 0)),
            pl.BlockSpec((TBG_, 128), lambda i: (i, 0)),
            pl.BlockSpec((TBG_, 128), lambda i: (i, 0)),
        ],
        out_shape=[
            jax.ShapeDtypeStruct((T_, 1), jnp.int32),
            jax.ShapeDtypeStruct((T_, 1), jnp.int32),
            jax.ShapeDtypeStruct((T_, 128), jnp.float32),
            jax.ShapeDtypeStruct((T_, 128), jnp.float32),
        ],
    )(x2d, wg_pad, bg_pad)


def _route_body(i1_ref, i2_ref, p0_ref, p1_ref, be_ref, bv_ref, gt_ref):
    lane = lax.broadcasted_iota(jnp.int32, (128, 128), 1)
    row = lax.broadcasted_iota(jnp.int32, (128, 128), 0)
    tril = jnp.where(lane < row, 1.0, 0.0).astype(jnp.bfloat16)
    triu = jnp.where(row < lane, 1.0, 0.0).astype(jnp.bfloat16)

    for g in range(32):
        i1c = i1_ref[pl.ds(g * 128, 128), :]
        i2c = i2_ref[pl.ds(g * 128, 128), :]
        o1 = jnp.where(i1c == lane, 1.0, 0.0)
        o2 = jnp.where(i2c == lane, 1.0, 0.0)
        gt_ref[pl.ds(g, 1), :] = jnp.sum(o1, axis=0, keepdims=True)
        gt_ref[pl.ds(32 + g, 1), :] = jnp.sum(o2, axis=0, keepdims=True)

    gt0 = gt_ref[pl.ds(0, 32), :]
    gt1 = gt_ref[pl.ds(32, 32), :]
    l32 = jnp.where(lax.broadcasted_iota(jnp.int32, (32, 32), 1)
                    < lax.broadcasted_iota(jnp.int32, (32, 32), 0),
                    1.0, 0.0).astype(jnp.bfloat16)
    gt0ex = jnp.dot(l32, gt0.astype(jnp.bfloat16),
                    preferred_element_type=jnp.float32)
    gt1ex = jnp.dot(l32, gt1.astype(jnp.bfloat16),
                    preferred_element_type=jnp.float32)
    c0 = jnp.sum(gt0, axis=0, keepdims=True)
    c1 = jnp.sum(gt1, axis=0, keepdims=True)
    counts = c0 + c1
    nb = (counts.astype(jnp.int32) + BT_ - 1) // BT_
    nbf = nb.astype(jnp.bfloat16)
    ps = jnp.dot(nbf, triu, preferred_element_type=jnp.float32) * float(BT_)

    for g in range(32):
        i1c = i1_ref[pl.ds(g * 128, 128), :]
        i2c = i2_ref[pl.ds(g * 128, 128), :]
        o1 = jnp.where(i1c == lane, 1.0, 0.0)
        o2 = jnp.where(i2c == lane, 1.0, 0.0)
        loc1 = jnp.dot(tril, o1.astype(jnp.bfloat16),
                       preferred_element_type=jnp.float32)
        loc2 = jnp.dot(tril, o2.astype(jnp.bfloat16),
                       preferred_element_type=jnp.float32)
        r0 = loc1 + gt0ex[g:g + 1, :]
        r1 = loc2 + gt1ex[g:g + 1, :] + c0
        p0c = jnp.sum(o1 * (r0 + ps), axis=1, keepdims=True)
        p1c = jnp.sum(o2 * (r1 + ps), axis=1, keepdims=True)
        p0_ref[pl.ds(g * 128, 128), :] = p0c.astype(jnp.int32)
        p1_ref[pl.ds(g * 128, 128), :] = p1c.astype(jnp.int32)

    csum = ps / float(BT_) + nb.astype(jnp.float32)
    bidx = lax.broadcasted_iota(jnp.int32, (MAXB_, 128), 0)
    lane8 = lax.broadcasted_iota(jnp.int32, (MAXB_, 128), 1) < E_
    ge = jnp.where(lane8 & (bidx >= csum.astype(jnp.int32)), 1, 0)
    be = jnp.sum(ge, axis=1, keepdims=True)
    tot = jnp.sum(jnp.where(lane8, nb, 0), axis=1, keepdims=True)
    bv = jnp.where(bidx[:, :1] < tot, 1, 0)
    be_ref[...] = jnp.where(bv == 1, be, 0)
    bv_ref[...] = bv


def _route(i1, i2):
    return pl.pallas_call(
        _route_body,
        grid=(1,),
        in_specs=[
            pl.BlockSpec((T_, 1), lambda i: (0, 0)),
            pl.BlockSpec((T_, 1), lambda i: (0, 0)),
        ],
        out_specs=[
            pl.BlockSpec((T_, 1), lambda i: (0, 0)),
            pl.BlockSpec((T_, 1), lambda i: (0, 0)),
            pl.BlockSpec((MAXB_, 1), lambda i: (0, 0)),
            pl.BlockSpec((MAXB_, 1), lambda i: (0, 0)),
        ],
        out_shape=[
            jax.ShapeDtypeStruct((T_, 1), jnp.int32),
            jax.ShapeDtypeStruct((T_, 1), jnp.int32),
            jax.ShapeDtypeStruct((MAXB_, 1), jnp.int32),
            jax.ShapeDtypeStruct((MAXB_, 1), jnp.int32),
        ],
        scratch_shapes=[pltpu.VMEM((64, 128), jnp.float32)],
    )(i1, i2)


@functools.lru_cache(maxsize=None)
def _sc_kernels():
    mesh = plsc.VectorSubcoreMesh(core_axis_name="c", subcore_axis_name="s",
                                  num_cores=2, num_subcores=16)

    @functools.partial(
        pl.kernel,
        out_type=[
            jax.ShapeDtypeStruct((NPAD_, D_), jnp.float32),
            jax.ShapeDtypeStruct((NPAD_, 128), jnp.float32),
        ],
        mesh=mesh,
        scratch_types=[
            pltpu.VMEM((NCD_, GD_), jnp.int32),
            pltpu.VMEM((NCD_, GD_), jnp.int32),
            pltpu.VMEM((GD_, D_), jnp.float32),
            pltpu.VMEM((GD_, D_), jnp.float32),
            pltpu.VMEM((GD_, 128), jnp.float32),
            pltpu.VMEM((GD_, 128), jnp.float32),
            pltpu.VMEM((GD_, 128), jnp.float32),
            pltpu.VMEM((GD_, 128), jnp.float32),
            pltpu.SemaphoreType.DMA,
            pltpu.SemaphoreType.DMA,
            pltpu.SemaphoreType.DMA,
            pltpu.SemaphoreType.DMA,
        ],
    )
    def _sc_dispatch(x_hbm, p0_hbm, p1_hbm, w1_hbm, w2_hbm, xs_hbm, ws_hbm,
                     p0_v, p1_v, xb0, xb1, wa0, wa1, wb0, wb1,
                     si0, si1, so0, so1):
        wid = lax.axis_index("s") * 2 + lax.axis_index("c")
        tok0 = wid * TPW_
        pltpu.sync_copy(p0_hbm.at[pl.ds(wid * NCD_, NCD_)], p0_v)
        pltpu.sync_copy(p1_hbm.at[pl.ds(wid * NCD_, NCD_)], p1_v)
        xb = (xb0, xb1)
        wa = (wa0, wa1)
        wb = (wb0, wb1)
        si = (si0, si1)
        so = (so0, so1)

        def start_in(c, b):
            sl = pl.ds(tok0 + c * GD_, GD_)
            return (pltpu.async_copy(x_hbm.at[sl], xb[b], si[b]),
                    pltpu.async_copy(w1_hbm.at[sl], wa[b], si[b]),
                    pltpu.async_copy(w2_hbm.at[sl], wb[b], si[b]))

        pend_in = start_in(0, 0)
        pend_sc = [None, None]
        for c in range(NCD_):
            b = c % 2
            for h in pend_in:
                h.wait()
            if c + 1 < NCD_:
                if pend_sc[1 - b] is not None:
                    for h in pend_sc[1 - b]:
                        h.wait()
                    pend_sc[1 - b] = None
                pend_in = start_in(c + 1, 1 - b)
            pend_sc[b] = (
                pltpu.async_copy(xb[b], xs_hbm.at[p0_v.at[c]], so[b]),
                pltpu.async_copy(xb[b], xs_hbm.at[p1_v.at[c]], so[b]),
                pltpu.async_copy(wa[b], ws_hbm.at[p0_v.at[c]], so[b]),
                pltpu.async_copy(wb[b], ws_hbm.at[p1_v.at[c]], so[b]),
            )
        for bb in (0, 1):
            if pend_sc[bb] is not None:
                for h in pend_sc[bb]:
                    h.wait()

    @functools.partial(
        pl.kernel,
        out_type=jax.ShapeDtypeStruct((T_, D_), jnp.float32),
        mesh=mesh,
        scratch_types=[
            pltpu.VMEM((NCH_, GC_), jnp.int32),
            pltpu.VMEM((NCH_, GC_), jnp.int32),
            pltpu.VMEM((GC_, D_), jnp.float32),
            pltpu.VMEM((GC_, D_), jnp.float32),
            pltpu.VMEM((GC_, D_), jnp.float32),
            pltpu.VMEM((GC_, D_), jnp.float32),
            pltpu.SemaphoreType.DMA,
            pltpu.SemaphoreType.DMA,
            pltpu.SemaphoreType.DMA,
            pltpu.SemaphoreType.DMA,
        ],
    )
    def _sc_combine(ys_hbm, p0_hbm, p1_hbm, out_hbm, p0_v, p1_v,
                    a0, a1, b0, b1, sg0, sg1, so0, so1):
        wid = lax.axis_index("s") * 2 + lax.axis_index("c")
        tok0 = wid * TPW_
        pltpu.sync_copy(p0_hbm.at[pl.ds(wid * NCH_, NCH_)], p0_v)
        pltpu.sync_copy(p1_hbm.at[pl.ds(wid * NCH_, NCH_)], p1_v)
        ab = (a0, a1)
        bb_ = (b0, b1)
        sg = (sg0, sg1)
        so = (so0, so1)

        def start_g(c, b):
            return (pltpu.async_copy(ys_hbm.at[p0_v.at[c]], ab[b], sg[b]),
                    pltpu.async_copy(ys_hbm.at[p1_v.at[c]], bb_[b], sg[b]))

        pend_g = start_g(0, 0)
        pend_o = [None, None]
        for c in range(NCH_):
            b = c % 2
            for h in pend_g:
                h.wait()
            if c + 1 < NCH_:
                if pend_o[1 - b] is not None:
                    pend_o[1 - b].wait()
                    pend_o[1 - b] = None
                pend_g = start_g(c + 1, 1 - b)

            def _row(r, carry, _ba=ab[b], _bb=bb_[b]):
                for col in range(0, D_, 16):
                    _ba[r, pl.ds(col, 16)] = (_ba[r, pl.ds(col, 16)]
                                              + _bb[r, pl.ds(col, 16)])
                return carry

            lax.fori_loop(0, GC_, _row, 0)
            pend_o[b] = pltpu.async_copy(
                ab[b], out_hbm.at[pl.ds(tok0 + c * GC_, GC_)], so[b])
        for z in (0, 1):
            if pend_o[z] is not None:
                pend_o[z].wait()

    return _sc_dispatch, _sc_combine


def _ffn_body(be_ref, bv_ref, xs_ref, w1_ref, b1_ref, w2_ref, b2_ref,
              ws_ref, out_ref):
    h = pl.program_id(1)
    i = pl.program_id(0)

    @pl.when(bv_ref[i] == 1)
    def _():
        @pl.when(h == 0)
        def _():
            out_ref[...] = jnp.zeros_like(out_ref)

        hh = jnp.dot(xs_ref[...].astype(jnp.bfloat16),
                     w1_ref[0].astype(jnp.bfloat16),
                     preferred_element_type=jnp.float32) + b1_ref[0, 0]
        hb = jnp.maximum(hh, 0.0).astype(jnp.bfloat16)
        out_ref[...] += jnp.dot(hb, w2_ref[0].astype(jnp.bfloat16),
                                preferred_element_type=jnp.float32)

        @pl.when(h == NH_ - 1)
        def _():
            out_ref[...] = (out_ref[...] + b2_ref[0]) * ws_ref[:, :1]


def _ffn(blk_e, blk_v, xs, W1, b1, W2, b2, ws16):
    grid_spec = pltpu.PrefetchScalarGridSpec(
        num_scalar_prefetch=2,
        grid=(MAXB_, NH_),
        in_specs=[
            pl.BlockSpec((BT_, D_), lambda i, h, be, bv: (i, 0)),
            pl.BlockSpec((1, D_, HT_),
                         lambda i, h, be, bv: (be[i], 0, h * bv[i])),
            pl.BlockSpec((1, 1, 1, HT_),
                         lambda i, h, be, bv: (be[i], h * bv[i], 0, 0)),
            pl.BlockSpec((1, HT_, D_),
                         lambda i, h, be, bv: (be[i], h * bv[i], 0)),
            pl.BlockSpec((1, 1, D_), lambda i, h, be, bv: (be[i], 0, 0)),
            pl.BlockSpec((BT_, 128), lambda i, h, be, bv: (i, 0)),
        ],
        out_specs=pl.BlockSpec((BT_, D_), lambda i, h, be, bv: (i, 0)),
    )
    return pl.pallas_call(
        _ffn_body,
        grid_spec=grid_spec,
        out_shape=jax.ShapeDtypeStruct((NPAD_, D_), jnp.float32),
    )(blk_e, blk_v, xs, W1, b1.reshape(E_, NH_, 1, HT_), W2,
      b2.reshape(E_, 1, D_), ws16)


def kernel(x, Wg, bg, W1, b1, W2, b2):
    x2d = x.reshape(T_, D_)
    wg_pad = jnp.zeros((D_, EP_), jnp.float32).at[:, :E_].set(Wg)
    bg_pad = jnp.full((1, EP_), -1e30, jnp.float32).at[0, :E_].set(bg)
    i1, i2, w1, w2 = _gate(x2d, wg_pad, bg_pad)
    p0, p1, blk_e, blk_v = _route(i1, i2)

    dispatch, combine = _sc_kernels()
    p0r = p0.reshape(T_ // GC_, GC_)
    p1r = p1.reshape(T_ // GC_, GC_)
    xs, ws16 = dispatch(x2d, p0.reshape(T_ // GD_, GD_),
                        p1.reshape(T_ // GD_, GD_), w1, w2)
    ys = _ffn(blk_e.reshape(MAXB_), blk_v.reshape(MAXB_), xs, W1, b1, W2,
              b2, ws16)
    out = combine(ys, p0r, p1r)
    return out.reshape(B_, S_, D_)
